# baseline (speedup 1.0000x reference)
import functools

import jax
import jax.numpy as jnp
from jax import lax
from jax.experimental import pallas as pl
from jax.experimental.pallas import tpu as pltpu
from jax.experimental.pallas import tpu_sc as plsc

H = 128
NC, NS = 2, 16
NW = NC * NS
CHUNK = 128
N_OUT = 5120
N_BUF = 5248
GARBAGE = 5184
BLK = 512


def _mm(a, b):
    return jnp.dot(a, b, preferred_element_type=jnp.float32)


def _enc_body(x_ref, w_ref, b_ref, o_ref):
    o_ref[...] = jnp.maximum(_mm(x_ref[...], w_ref[...]) + b_ref[...], 0.0)


def _encode(x, w, b):
    n = x.shape[0]
    return pl.pallas_call(
        _enc_body,
        grid=(pl.cdiv(n, BLK),),
        in_specs=[
            pl.BlockSpec((BLK, H), lambda i: (i, 0)),
            pl.BlockSpec((H, H), lambda i: (0, 0)),
            pl.BlockSpec((1, H), lambda i: (0, 0)),
        ],
        out_specs=pl.BlockSpec((BLK, H), lambda i: (i, 0)),
        out_shape=jax.ShapeDtypeStruct((n, H), jnp.float32),
    )(x, w, b.reshape(1, H))


def _joint_body(ab_ref, aj_ref, af_ref, x_ref, w_ref, wr_ref, b_ref, o_ref):
    acc = _mm(ab_ref[0] + ab_ref[1], w_ref[0])
    acc += _mm(aj_ref[0] + aj_ref[1], w_ref[1])
    acc += _mm(af_ref[0] + af_ref[1], w_ref[2])
    wr = wr_ref[0] + wr_ref[1] + wr_ref[2]
    acc += _mm(x_ref[...], wr)
    acc += b_ref[0:1] + b_ref[1:2] + b_ref[2:3]
    o_ref[...] = jnp.maximum(acc, 0.0)


def _combine_joint(ab, aj, af, x, ws, wrs, bs, n):
    return pl.pallas_call(
        _joint_body,
        grid=(pl.cdiv(n, BLK),),
        in_specs=[
            pl.BlockSpec((2, BLK, H), lambda i: (0, i, 0)),
            pl.BlockSpec((2, BLK, H), lambda i: (0, i, 0)),
            pl.BlockSpec((2, BLK, H), lambda i: (0, i, 0)),
            pl.BlockSpec((BLK, H), lambda i: (i, 0)),
            pl.BlockSpec((3, H, H), lambda i: (0, 0, 0)),
            pl.BlockSpec((3, H, H), lambda i: (0, 0, 0)),
            pl.BlockSpec((3, H), lambda i: (0, 0)),
        ],
        out_specs=pl.BlockSpec((BLK, H), lambda i: (i, 0)),
        out_shape=jax.ShapeDtypeStruct((n, H), jnp.float32),
    )(ab, aj, af, x, ws, wrs, bs)


def _foot_body(a_ref, x_ref, w_ref, wr_ref, b_ref, o_ref):
    acc = _mm(a_ref[0] + a_ref[1], w_ref[...])
    acc += _mm(x_ref[...], wr_ref[...])
    acc += b_ref[...]
    o_ref[...] = jnp.maximum(acc, 0.0)


def _combine_foot(a, x, w, wr, b, n):
    return pl.pallas_call(
        _foot_body,
        grid=(pl.cdiv(n, BLK),),
        in_specs=[
            pl.BlockSpec((2, BLK, H), lambda i: (0, i, 0)),
            pl.BlockSpec((BLK, H), lambda i: (i, 0)),
            pl.BlockSpec((H, H), lambda i: (0, 0)),
            pl.BlockSpec((H, H), lambda i: (0, 0)),
            pl.BlockSpec((1, H), lambda i: (0, 0)),
        ],
        out_specs=pl.BlockSpec((BLK, H), lambda i: (i, 0)),
        out_shape=jax.ShapeDtypeStruct((n, H), jnp.float32),
    )(a, x, w, wr, b.reshape(1, H))


def _foot_dec_body(a_ref, x_ref, w_ref, wr_ref, b_ref, wd_ref, bd_ref, o_ref):
    acc = _mm(a_ref[0] + a_ref[1], w_ref[...])
    acc += _mm(x_ref[...], wr_ref[...])
    acc += b_ref[...]
    h = jnp.maximum(acc, 0.0)
    o_ref[...] = _mm(h, wd_ref[...]) + bd_ref[...]


def _combine_foot_dec(a, x, w, wr, b, wd, bd, n):
    return pl.pallas_call(
        _foot_dec_body,
        grid=(pl.cdiv(n, BLK),),
        in_specs=[
            pl.BlockSpec((2, BLK, H), lambda i: (0, i, 0)),
            pl.BlockSpec((BLK, H), lambda i: (i, 0)),
            pl.BlockSpec((H, H), lambda i: (0, 0)),
            pl.BlockSpec((H, H), lambda i: (0, 0)),
            pl.BlockSpec((1, H), lambda i: (0, 0)),
            pl.BlockSpec((H, H), lambda i: (0, 0)),
            pl.BlockSpec((1, H), lambda i: (0, 0)),
        ],
        out_specs=pl.BlockSpec((BLK, H), lambda i: (i, 0)),
        out_shape=jax.ShapeDtypeStruct((n, H), jnp.float32),
    )(a, x, w, wr, b.reshape(1, H), wd, bd.reshape(1, H))


G = 2


def _zero_vmem_slab(ref):
    zero = jnp.zeros((16,), jnp.float32)

    def row(i, _):
        def col(j, _):
            ref[i, pl.ds(j * 16, 16)] = zero
            return 0
        return lax.fori_loop(0, H // 16, col, 0)

    lax.fori_loop(0, CHUNK, row, 0)


def _zero_spmem(buf, sid, zslab):
    off = sid * (N_BUF // NS)
    pltpu.sync_copy(zslab.at[pl.ds(0, CHUNK)], buf.at[pl.ds(off, CHUNK)])
    pltpu.sync_copy(zslab.at[pl.ds(0, CHUNK)], buf.at[pl.ds(off + CHUNK, CHUNK)])
    pltpu.sync_copy(zslab.at[pl.ds(0, N_BUF // NS - 2 * CHUNK)],
                    buf.at[pl.ds(off + 2 * CHUNK, N_BUF // NS - 2 * CHUNK)])


def _clamp_dst(di, n_chunks):
    def body(i, _):
        for c in range(H // 16):
            v = di[i, pl.ds(c * 16, 16)]
            di[i, pl.ds(c * 16, 16)] = jnp.where(v < 5000, v, GARBAGE)
        return 0
    lax.fori_loop(0, n_chunks, body, 0)


def _pipeline(si, di, table, buf, rows0, rows1, sg0, sg1, ss0, ss1, w,
              n_chunks):
    base = w * n_chunks
    n_pairs = n_chunks // (2 * G)

    def g_start(rows, sem, row0):
        for j in range(G):
            pltpu.async_copy(table.at[si.at[row0 + j]], rows.at[j], sem)

    def g_wait(rows, sem):
        for j in range(G):
            pltpu.make_async_copy(table.at[si.at[0]], rows.at[j], sem).wait()

    def s_start(rows, sem, row0):
        for j in range(G):
            pltpu.async_copy(rows.at[j], buf.at[di.at[row0 + j]], sem,
                             add=True)

    def s_wait(rows, sem):
        for j in range(G):
            pltpu.make_async_copy(rows.at[j], buf.at[di.at[0]], sem).wait()

    g_start(rows0, sg0, base)

    def body(p, _):
        row_a = base + p * 2 * G
        row_b = row_a + G
        g_wait(rows0, sg0)

        @pl.when(p > 0)
        def _():
            s_wait(rows1, ss1)

        g_start(rows1, sg1, row_b)
        s_start(rows0, ss0, row_a)
        g_wait(rows1, sg1)
        s_wait(rows0, ss0)

        @pl.when(p < n_pairs - 1)
        def _():
            g_start(rows0, sg0, row_a + 2 * G)

        s_start(rows1, ss1, row_b)
        return 0

    lax.fori_loop(0, n_pairs, body, 0)
    s_wait(rows1, ss1)


def _flush(buf, out, cid, sid):
    rows_per = N_OUT // NS
    off = sid * rows_per
    pltpu.sync_copy(buf.at[pl.ds(off, rows_per)],
                    out.at[cid, pl.ds(off, rows_per)])


_SC_MESH = plsc.VectorSubcoreMesh(core_axis_name="c", subcore_axis_name="s",
                                  num_cores=NC, num_subcores=NS)


def _sc_layer0(xb, xj, xf, sb, db, sj, dj, sf, df, sjf, djf,
               nb_chunks, nj_chunks, nf_chunks, njf_chunks):
    agg_ty = jax.ShapeDtypeStruct((NC, N_OUT, H), jnp.float32)

    n_idx = max(nb_chunks, nj_chunks, nf_chunks, njf_chunks)

    @functools.partial(
        pl.kernel,
        out_type=(agg_ty, agg_ty, agg_ty, agg_ty),
        mesh=_SC_MESH,
        scratch_types=[
            pltpu.VMEM_SHARED((N_BUF, H), jnp.float32),
            pltpu.VMEM((n_idx, CHUNK), jnp.int32),
            pltpu.VMEM((n_idx, CHUNK), jnp.int32),
            pltpu.VMEM((G, CHUNK, H), jnp.float32),
            pltpu.VMEM((G, CHUNK, H), jnp.float32),
            pltpu.SemaphoreType.DMA,
            pltpu.SemaphoreType.DMA,
            pltpu.SemaphoreType.DMA,
            pltpu.SemaphoreType.DMA,
        ],
    )
    def k(xb_h, xj_h, xf_h, sb_h, db_h, sj_h, dj_h, sf_h, df_h, sjf_h, djf_h,
          ob, oj, of_, ojf, buf, si, di, rows0, rows1, sg0, sg1, ss0, ss1):
        cid = lax.axis_index("c")
        sid = lax.axis_index("s")
        w = sid * NC + cid
        phases = [
            (sb_h, db_h, xb_h, ob, nb_chunks, False),
            (sj_h, dj_h, xj_h, oj, nj_chunks, True),
            (sf_h, df_h, xf_h, of_, nf_chunks, False),
            (sjf_h, djf_h, xj_h, ojf, njf_chunks, False),
        ]
        for s_h, d_h, table, out, n_chunks, clamp in phases:
            pltpu.sync_copy(s_h.at[w], si.at[pl.ds(0, n_chunks)])
            pltpu.sync_copy(d_h.at[w], di.at[pl.ds(0, n_chunks)])
            if clamp:
                _clamp_dst(di, n_chunks)
            _zero_vmem_slab(rows0.at[0])
            _zero_spmem(buf, sid, rows0.at[0])
            plsc.subcore_barrier()
            _pipeline(si, di, table, buf, rows0, rows1, sg0, sg1, ss0, ss1,
                      w=0, n_chunks=n_chunks)
            plsc.subcore_barrier()
            _flush(buf, out, cid, sid)
            plsc.subcore_barrier()

    return k(xb, xj, xf, sb, db, sj, dj, sf, df, sjf, djf)


def _sc_layer1(xj1, sjf, djf, njf_chunks):
    agg_ty = jax.ShapeDtypeStruct((NC, N_OUT, H), jnp.float32)

    @functools.partial(
        pl.kernel,
        out_type=agg_ty,
        mesh=_SC_MESH,
        scratch_types=[
            pltpu.VMEM_SHARED((N_BUF, H), jnp.float32),
            pltpu.VMEM((njf_chunks, CHUNK), jnp.int32),
            pltpu.VMEM((njf_chunks, CHUNK), jnp.int32),
            pltpu.VMEM((G, CHUNK, H), jnp.float32),
            pltpu.VMEM((G, CHUNK, H), jnp.float32),
            pltpu.SemaphoreType.DMA,
            pltpu.SemaphoreType.DMA,
            pltpu.SemaphoreType.DMA,
            pltpu.SemaphoreType.DMA,
        ],
    )
    def k(xj_h, s_h, d_h, out, buf, si, di, rows0, rows1, sg0, sg1, ss0, ss1):
        cid = lax.axis_index("c")
        sid = lax.axis_index("s")
        w = sid * NC + cid
        pltpu.sync_copy(s_h.at[w], si)
        pltpu.sync_copy(d_h.at[w], di)
        _zero_vmem_slab(rows0.at[0])
        _zero_spmem(buf, sid, rows0.at[0])
        plsc.subcore_barrier()
        _pipeline(si, di, xj_h, buf, rows0, rows1, sg0, sg1, ss0, ss1,
                  w=0, n_chunks=njf_chunks)
        plsc.subcore_barrier()
        _flush(buf, out, cid, sid)

    return k(xj1, sjf, djf)


def _pad_edges(ei, n_chunks):
    e_pad = NW * n_chunks * CHUNK
    pad = e_pad - ei.shape[1]
    s = jnp.concatenate([ei[0], jnp.zeros((pad,), jnp.int32)])
    d = jnp.concatenate([ei[1], jnp.full((pad,), GARBAGE, jnp.int32)])
    return s.reshape(NW, n_chunks, CHUNK), d.reshape(NW, n_chunks, CHUNK)


def _n_chunks(e):
    per = pl.cdiv(e, NW * CHUNK)
    return ((per + 2 * G - 1) // (2 * G)) * (2 * G)


def kernel(x_base, x_joint, x_foot, ei_b2j, ei_j2b, ei_j2j, ei_j2f, ei_f2j,
           W_enc, b_enc, W_rel, b_rel, W_root, W_dec, b_dec):
    del ei_j2b

    nb = _n_chunks(ei_b2j.shape[1])
    nj = _n_chunks(ei_j2j.shape[1])
    nf = _n_chunks(ei_f2j.shape[1])
    njf = _n_chunks(ei_j2f.shape[1])
    sb, db = _pad_edges(ei_b2j, nb)
    sj, dj = _pad_edges(ei_j2j, nj)
    sf, df = _pad_edges(ei_f2j, nf)
    sjf, djf = _pad_edges(ei_j2f, njf)

    xb0 = _encode(x_base, W_enc[0], b_enc[0])
    xj0 = _encode(x_joint, W_enc[1], b_enc[1])
    xf0 = _encode(x_foot, W_enc[2], b_enc[2])

    a_b2j, a_j2j, a_f2j, a_j2f = _sc_layer0(
        xb0, xj0, xf0, sb, db, sj, dj, sf, df, sjf, djf, nb, nj, nf, njf)

    ws_j = jnp.stack([W_rel[0, 0], W_rel[0, 2], W_rel[0, 4]])
    wrs_j = jnp.stack([W_root[0, 0], W_root[0, 2], W_root[0, 4]])
    bs_j = jnp.stack([b_rel[0, 0], b_rel[0, 2], b_rel[0, 4]])
    xj1 = _combine_joint(a_b2j, a_j2j, a_f2j, xj0, ws_j, wrs_j, bs_j, 5000)
    xf1 = _combine_foot(a_j2f, xf0, W_rel[0, 3], W_root[0, 3], b_rel[0, 3],
                        5000)

    a2 = _sc_layer1(xj1, sjf, djf, njf)

    wd_pad = jnp.zeros((H, H), jnp.float32).at[:, 0].set(W_dec[:, 0])
    bd_pad = jnp.zeros((H,), jnp.float32).at[0].set(b_dec[0])
    out = _combine_foot_dec(a2, xf1, W_rel[1, 3], W_root[1, 3], b_rel[1, 3],
                            wd_pad, bd_pad, 5000)
    return out[:, 0:1]


# speedup vs baseline: 1.0346x; 1.0346x over previous
import functools

import jax
import jax.numpy as jnp
from jax import lax
from jax.experimental import pallas as pl
from jax.experimental.pallas import tpu as pltpu
from jax.experimental.pallas import tpu_sc as plsc

H = 128
NC, NS = 2, 16
NW = NC * NS
CHUNK = 128
N_OUT = 5120
N_BUF = 5248
GARBAGE = 5184
BLK = 512


def _mm(a, b):
    return jnp.dot(a, b, preferred_element_type=jnp.float32)


def _enc_body(x_ref, w_ref, b_ref, o_ref):
    o_ref[...] = jnp.maximum(_mm(x_ref[...], w_ref[...]) + b_ref[...], 0.0)


def _encode(x, w, b):
    n = x.shape[0]
    return pl.pallas_call(
        _enc_body,
        grid=(pl.cdiv(n, BLK),),
        in_specs=[
            pl.BlockSpec((BLK, H), lambda i: (i, 0)),
            pl.BlockSpec((H, H), lambda i: (0, 0)),
            pl.BlockSpec((1, H), lambda i: (0, 0)),
        ],
        out_specs=pl.BlockSpec((BLK, H), lambda i: (i, 0)),
        out_shape=jax.ShapeDtypeStruct((n, H), jnp.float32),
    )(x, w, b.reshape(1, H))


def _joint_body(ab_ref, aj_ref, af_ref, x_ref, w_ref, wr_ref, b_ref, o_ref):
    acc = _mm(ab_ref[0] + ab_ref[1], w_ref[0])
    acc += _mm(aj_ref[0] + aj_ref[1], w_ref[1])
    acc += _mm(af_ref[0] + af_ref[1], w_ref[2])
    wr = wr_ref[0] + wr_ref[1] + wr_ref[2]
    acc += _mm(x_ref[...], wr)
    acc += b_ref[0:1] + b_ref[1:2] + b_ref[2:3]
    o_ref[...] = jnp.maximum(acc, 0.0)


def _combine_joint(ab, aj, af, x, ws, wrs, bs, n):
    return pl.pallas_call(
        _joint_body,
        grid=(pl.cdiv(n, BLK),),
        in_specs=[
            pl.BlockSpec((2, BLK, H), lambda i: (0, i, 0)),
            pl.BlockSpec((2, BLK, H), lambda i: (0, i, 0)),
            pl.BlockSpec((2, BLK, H), lambda i: (0, i, 0)),
            pl.BlockSpec((BLK, H), lambda i: (i, 0)),
            pl.BlockSpec((3, H, H), lambda i: (0, 0, 0)),
            pl.BlockSpec((3, H, H), lambda i: (0, 0, 0)),
            pl.BlockSpec((3, H), lambda i: (0, 0)),
        ],
        out_specs=pl.BlockSpec((BLK, H), lambda i: (i, 0)),
        out_shape=jax.ShapeDtypeStruct((n, H), jnp.float32),
    )(ab, aj, af, x, ws, wrs, bs)


def _foot_body(a_ref, x_ref, w_ref, wr_ref, b_ref, o_ref):
    acc = _mm(a_ref[0] + a_ref[1], w_ref[...])
    acc += _mm(x_ref[...], wr_ref[...])
    acc += b_ref[...]
    o_ref[...] = jnp.maximum(acc, 0.0)


def _combine_foot(a, x, w, wr, b, n):
    return pl.pallas_call(
        _foot_body,
        grid=(pl.cdiv(n, BLK),),
        in_specs=[
            pl.BlockSpec((2, BLK, H), lambda i: (0, i, 0)),
            pl.BlockSpec((BLK, H), lambda i: (i, 0)),
            pl.BlockSpec((H, H), lambda i: (0, 0)),
            pl.BlockSpec((H, H), lambda i: (0, 0)),
            pl.BlockSpec((1, H), lambda i: (0, 0)),
        ],
        out_specs=pl.BlockSpec((BLK, H), lambda i: (i, 0)),
        out_shape=jax.ShapeDtypeStruct((n, H), jnp.float32),
    )(a, x, w, wr, b.reshape(1, H))


def _foot_dec_body(a_ref, x_ref, w_ref, wr_ref, b_ref, wd_ref, bd_ref, o_ref):
    acc = _mm(a_ref[0] + a_ref[1], w_ref[...])
    acc += _mm(x_ref[...], wr_ref[...])
    acc += b_ref[...]
    h = jnp.maximum(acc, 0.0)
    o_ref[...] = _mm(h, wd_ref[...]) + bd_ref[...]


def _combine_foot_dec(a, x, w, wr, b, wd, bd, n):
    return pl.pallas_call(
        _foot_dec_body,
        grid=(pl.cdiv(n, BLK),),
        in_specs=[
            pl.BlockSpec((2, BLK, H), lambda i: (0, i, 0)),
            pl.BlockSpec((BLK, H), lambda i: (i, 0)),
            pl.BlockSpec((H, H), lambda i: (0, 0)),
            pl.BlockSpec((H, H), lambda i: (0, 0)),
            pl.BlockSpec((1, H), lambda i: (0, 0)),
            pl.BlockSpec((H, H), lambda i: (0, 0)),
            pl.BlockSpec((1, H), lambda i: (0, 0)),
        ],
        out_specs=pl.BlockSpec((BLK, H), lambda i: (i, 0)),
        out_shape=jax.ShapeDtypeStruct((n, H), jnp.float32),
    )(a, x, w, wr, b.reshape(1, H), wd, bd.reshape(1, H))


def _zero_slab(ref):
    zero = jnp.zeros((16,), jnp.float32)

    def row(i, _):
        for j in range(H // 16):
            ref[i, pl.ds(j * 16, 16)] = zero
        return 0

    lax.fori_loop(0, CHUNK, row, 0)


def _zero_buf(buf, sid, zslab):
    per = N_BUF // NS
    off = sid * per
    done = 0
    while done < per:
        step = min(CHUNK, per - done)
        pltpu.sync_copy(zslab.at[pl.ds(0, step)],
                        buf.at[pl.ds(off + done, step)])
        done += step


def _pipeline(s1d, d1d, table, buf, slots, w, n_chunks):
    base = w * n_chunks
    n_trips = n_chunks // 3

    def i_start(t, row):
        ts, td, _, si, _, _ = t
        off = pl.multiple_of((base + row) * CHUNK, 8)
        pltpu.async_copy(s1d.at[pl.ds(off, CHUNK)], ts, si)
        pltpu.async_copy(d1d.at[pl.ds(off, CHUNK)], td, si)

    def i_wait(t):
        ts, td, _, si, _, _ = t
        pltpu.make_async_copy(s1d.at[pl.ds(0, CHUNK)], ts, si).wait()
        pltpu.make_async_copy(d1d.at[pl.ds(0, CHUNK)], td, si).wait()

    def g_start(t):
        ts, _, rows, _, sg, _ = t
        pltpu.async_copy(table.at[ts], rows, sg)

    def g_wait(t):
        ts, _, rows, _, sg, _ = t
        pltpu.make_async_copy(table.at[ts], rows, sg).wait()

    def s_start(t):
        _, td, rows, _, _, ss = t
        pltpu.async_copy(rows, buf.at[td], ss, add=True)

    def s_wait(t):
        _, td, rows, _, _, ss = t
        pltpu.make_async_copy(rows, buf.at[td], ss).wait()

    i_start(slots[0], 0)
    i_start(slots[1], 1)
    i_wait(slots[0])
    g_start(slots[0])
    i_wait(slots[1])
    g_start(slots[1])

    def body(q, _):
        for j in range(3):
            g = q * 3 + j
            t = slots[j]
            t2 = slots[(j + 2) % 3]
            g_wait(t)
            if j == 0:
                @pl.when(q > 0)
                def _():
                    s_wait(t2)
            else:
                s_wait(t2)
            if j == 0:
                i_start(t2, g + 2)
                s_start(t)
                i_wait(t2)
                g_start(t2)
            else:
                @pl.when(q < n_trips - 1)
                def _():
                    i_start(t2, g + 2)

                s_start(t)

                @pl.when(q < n_trips - 1)
                def _():
                    i_wait(t2)
                    g_start(t2)
        return 0

    lax.fori_loop(0, n_trips, body, 0)
    s_wait(slots[2])


def _flush(buf, out, cid, sid):
    rows_per = N_OUT // NS
    off = sid * rows_per
    pltpu.sync_copy(buf.at[pl.ds(off, rows_per)],
                    out.at[cid, pl.ds(off, rows_per)])


_SC_MESH = plsc.VectorSubcoreMesh(core_axis_name="c", subcore_axis_name="s",
                                  num_cores=NC, num_subcores=NS)

_SC_SCRATCH = [
    pltpu.VMEM_SHARED((N_BUF, H), jnp.float32),
    pltpu.VMEM((CHUNK,), jnp.int32),
    pltpu.VMEM((CHUNK,), jnp.int32),
    pltpu.VMEM((CHUNK,), jnp.int32),
    pltpu.VMEM((CHUNK,), jnp.int32),
    pltpu.VMEM((CHUNK,), jnp.int32),
    pltpu.VMEM((CHUNK,), jnp.int32),
    pltpu.VMEM((CHUNK, H), jnp.float32),
    pltpu.VMEM((CHUNK, H), jnp.float32),
    pltpu.VMEM((CHUNK, H), jnp.float32),
    pltpu.SemaphoreType.DMA,
    pltpu.SemaphoreType.DMA,
    pltpu.SemaphoreType.DMA,
    pltpu.SemaphoreType.DMA,
    pltpu.SemaphoreType.DMA,
    pltpu.SemaphoreType.DMA,
    pltpu.SemaphoreType.DMA,
    pltpu.SemaphoreType.DMA,
    pltpu.SemaphoreType.DMA,
]


def _make_slots(refs):
    (ts0, td0, ts1, td1, ts2, td2, rows0, rows1, rows2,
     si0, si1, si2, sg0, sg1, sg2, ss0, ss1, ss2) = refs
    return [(ts0, td0, rows0, si0, sg0, ss0),
            (ts1, td1, rows1, si1, sg1, ss1),
            (ts2, td2, rows2, si2, sg2, ss2)]


def _sc_layer0(xb, xj, xf, sb, db, sj, dj, sf, df, sjf, djf,
               nb_chunks, nj_chunks, nf_chunks, njf_chunks):
    agg_ty = jax.ShapeDtypeStruct((NC, N_OUT, H), jnp.float32)

    @functools.partial(
        pl.kernel,
        out_type=(agg_ty, agg_ty, agg_ty, agg_ty),
        mesh=_SC_MESH,
        scratch_types=_SC_SCRATCH,
    )
    def k(xb_h, xj_h, xf_h, sb_h, db_h, sj_h, dj_h, sf_h, df_h, sjf_h, djf_h,
          ob, oj, of_, ojf, buf, *refs):
        cid = lax.axis_index("c")
        sid = lax.axis_index("s")
        w = sid * NC + cid
        slots = _make_slots(refs)
        zslab = slots[0][2]

        def run(s_h, d_h, table, out, n_chunks):
            _zero_slab(zslab)
            _zero_buf(buf, sid, zslab)
            plsc.subcore_barrier()
            _pipeline(s_h, d_h, table, buf, slots, w, n_chunks)
            plsc.subcore_barrier()
            _flush(buf, out, cid, sid)
            plsc.subcore_barrier()

        run(sb_h, db_h, xb_h, ob, nb_chunks)
        run(sf_h, df_h, xf_h, of_, nf_chunks)
        run(sj_h, dj_h, xj_h, oj, nj_chunks)
        run(sjf_h, djf_h, xj_h, ojf, njf_chunks)

    return k(xb, xj, xf, sb, db, sj, dj, sf, df, sjf, djf)


def _sc_layer1(xj1, sjf, djf, njf_chunks):
    agg_ty = jax.ShapeDtypeStruct((NC, N_OUT, H), jnp.float32)

    @functools.partial(
        pl.kernel,
        out_type=agg_ty,
        mesh=_SC_MESH,
        scratch_types=_SC_SCRATCH,
    )
    def k(xj_h, s_h, d_h, out, buf, *refs):
        cid = lax.axis_index("c")
        sid = lax.axis_index("s")
        w = sid * NC + cid
        slots = _make_slots(refs)
        zslab = slots[0][2]
        _zero_slab(zslab)
        _zero_buf(buf, sid, zslab)
        plsc.subcore_barrier()
        _pipeline(s_h, d_h, xj_h, buf, slots, w, njf_chunks)
        plsc.subcore_barrier()
        _flush(buf, out, cid, sid)

    return k(xj1, sjf, djf)


def _pad_edges(ei, n_chunks, dst_clamp=False):
    e_pad = NW * n_chunks * CHUNK
    pad = e_pad - ei.shape[1]
    d = jnp.where(ei[1] < 5000, ei[1], GARBAGE) if dst_clamp else ei[1]
    s = jnp.concatenate([ei[0], jnp.zeros((pad,), jnp.int32)])
    d = jnp.concatenate([d, jnp.full((pad,), GARBAGE, jnp.int32)])
    return s, d


def _n_chunks(e):
    per = pl.cdiv(e, NW * CHUNK)
    return ((per + 2) // 3) * 3


def kernel(x_base, x_joint, x_foot, ei_b2j, ei_j2b, ei_j2j, ei_j2f, ei_f2j,
           W_enc, b_enc, W_rel, b_rel, W_root, W_dec, b_dec):
    del ei_j2b

    nb = _n_chunks(ei_b2j.shape[1])
    nj = _n_chunks(ei_j2j.shape[1])
    nf = _n_chunks(ei_f2j.shape[1])
    njf = _n_chunks(ei_j2f.shape[1])
    sb, db = _pad_edges(ei_b2j, nb)
    sj, dj = _pad_edges(ei_j2j, nj, dst_clamp=True)
    sf, df = _pad_edges(ei_f2j, nf)
    sjf, djf = _pad_edges(ei_j2f, njf)

    xb0 = _encode(x_base, W_enc[0], b_enc[0])
    xj0 = _encode(x_joint, W_enc[1], b_enc[1])
    xf0 = _encode(x_foot, W_enc[2], b_enc[2])

    a_b2j, a_j2j, a_f2j, a_j2f = _sc_layer0(
        xb0, xj0, xf0, sb, db, sj, dj, sf, df, sjf, djf, nb, nj, nf, njf)

    ws_j = jnp.stack([W_rel[0, 0], W_rel[0, 2], W_rel[0, 4]])
    wrs_j = jnp.stack([W_root[0, 0], W_root[0, 2], W_root[0, 4]])
    bs_j = jnp.stack([b_rel[0, 0], b_rel[0, 2], b_rel[0, 4]])
    xj1 = _combine_joint(a_b2j, a_j2j, a_f2j, xj0, ws_j, wrs_j, bs_j, 5000)
    xf1 = _combine_foot(a_j2f, xf0, W_rel[0, 3], W_root[0, 3], b_rel[0, 3],
                        5000)

    a2 = _sc_layer1(xj1, sjf, djf, njf)

    wd_pad = jnp.zeros((H, H), jnp.float32).at[:, 0].set(W_dec[:, 0])
    bd_pad = jnp.zeros((H,), jnp.float32).at[0].set(b_dec[0])
    out = _combine_foot_dec(a2, xf1, W_rel[1, 3], W_root[1, 3], b_rel[1, 3],
                            wd_pad, bd_pad, 5000)
    return out[:, 0:1]


# speedup vs baseline: 1.3517x; 1.3066x over previous
import functools

import jax
import jax.numpy as jnp
from jax import lax
from jax.experimental import pallas as pl
from jax.experimental.pallas import tpu as pltpu
from jax.experimental.pallas import tpu_sc as plsc

H = 128
NC, NS = 2, 16
NW = NC * NS
CHUNK = 128
N_OUT = 5120
N_BUF = 5248
GARBAGE = 5184
BLK = 512


def _mm(a, b):
    return jnp.dot(a, b, preferred_element_type=jnp.float32)


def _enc_body(x_ref, w_ref, b_ref, o_ref):
    o_ref[...] = jnp.maximum(_mm(x_ref[...], w_ref[...]) + b_ref[...], 0.0)


def _encode(x, w, b):
    n = x.shape[0]
    return pl.pallas_call(
        _enc_body,
        grid=(pl.cdiv(n, BLK),),
        in_specs=[
            pl.BlockSpec((BLK, H), lambda i: (i, 0)),
            pl.BlockSpec((H, H), lambda i: (0, 0)),
            pl.BlockSpec((1, H), lambda i: (0, 0)),
        ],
        out_specs=pl.BlockSpec((BLK, H), lambda i: (i, 0)),
        out_shape=jax.ShapeDtypeStruct((n, H), jnp.float32),
    )(x, w, b.reshape(1, H))


def _joint_body(ab_ref, aj_ref, af_ref, x_ref, w_ref, wr_ref, b_ref, o_ref):
    acc = _mm(ab_ref[0] + ab_ref[1], w_ref[0])
    acc += _mm(aj_ref[0] + aj_ref[1], w_ref[1])
    acc += _mm(af_ref[0] + af_ref[1], w_ref[2])
    wr = wr_ref[0] + wr_ref[1] + wr_ref[2]
    acc += _mm(x_ref[...], wr)
    acc += b_ref[0:1] + b_ref[1:2] + b_ref[2:3]
    o_ref[...] = jnp.maximum(acc, 0.0)


def _combine_joint(ab, aj, af, x, ws, wrs, bs, n):
    return pl.pallas_call(
        _joint_body,
        grid=(pl.cdiv(n, BLK),),
        in_specs=[
            pl.BlockSpec((2, BLK, H), lambda i: (0, i, 0)),
            pl.BlockSpec((2, BLK, H), lambda i: (0, i, 0)),
            pl.BlockSpec((2, BLK, H), lambda i: (0, i, 0)),
            pl.BlockSpec((BLK, H), lambda i: (i, 0)),
            pl.BlockSpec((3, H, H), lambda i: (0, 0, 0)),
            pl.BlockSpec((3, H, H), lambda i: (0, 0, 0)),
            pl.BlockSpec((3, H), lambda i: (0, 0)),
        ],
        out_specs=pl.BlockSpec((BLK, H), lambda i: (i, 0)),
        out_shape=jax.ShapeDtypeStruct((n, H), jnp.float32),
    )(ab, aj, af, x, ws, wrs, bs)


def _foot_body(a_ref, x_ref, w_ref, wr_ref, b_ref, o_ref):
    acc = _mm(a_ref[0] + a_ref[1], w_ref[...])
    acc += _mm(x_ref[...], wr_ref[...])
    acc += b_ref[...]
    o_ref[...] = jnp.maximum(acc, 0.0)


def _combine_foot(a, x, w, wr, b, n):
    return pl.pallas_call(
        _foot_body,
        grid=(pl.cdiv(n, BLK),),
        in_specs=[
            pl.BlockSpec((2, BLK, H), lambda i: (0, i, 0)),
            pl.BlockSpec((BLK, H), lambda i: (i, 0)),
            pl.BlockSpec((H, H), lambda i: (0, 0)),
            pl.BlockSpec((H, H), lambda i: (0, 0)),
            pl.BlockSpec((1, H), lambda i: (0, 0)),
        ],
        out_specs=pl.BlockSpec((BLK, H), lambda i: (i, 0)),
        out_shape=jax.ShapeDtypeStruct((n, H), jnp.float32),
    )(a, x, w, wr, b.reshape(1, H))


def _foot_dec_body(a_ref, x_ref, w_ref, wr_ref, b_ref, wd_ref, bd_ref, o_ref):
    acc = _mm(a_ref[0] + a_ref[1], w_ref[...])
    acc += _mm(x_ref[...], wr_ref[...])
    acc += b_ref[...]
    h = jnp.maximum(acc, 0.0)
    o_ref[...] = _mm(h, wd_ref[...]) + bd_ref[...]


def _combine_foot_dec(a, x, w, wr, b, wd, bd, n):
    return pl.pallas_call(
        _foot_dec_body,
        grid=(pl.cdiv(n, BLK),),
        in_specs=[
            pl.BlockSpec((2, BLK, H), lambda i: (0, i, 0)),
            pl.BlockSpec((BLK, H), lambda i: (i, 0)),
            pl.BlockSpec((H, H), lambda i: (0, 0)),
            pl.BlockSpec((H, H), lambda i: (0, 0)),
            pl.BlockSpec((1, H), lambda i: (0, 0)),
            pl.BlockSpec((H, H), lambda i: (0, 0)),
            pl.BlockSpec((1, H), lambda i: (0, 0)),
        ],
        out_specs=pl.BlockSpec((BLK, H), lambda i: (i, 0)),
        out_shape=jax.ShapeDtypeStruct((n, H), jnp.float32),
    )(a, x, w, wr, b.reshape(1, H), wd, bd.reshape(1, H))


def _zero_slab(ref):
    zero = jnp.zeros((16,), jnp.float32)

    def row(i, _):
        for j in range(H // 16):
            ref[i, pl.ds(j * 16, 16)] = zero
        return 0

    lax.fori_loop(0, CHUNK, row, 0)


def _zero_buf(buf, sid, zslab):
    per = N_BUF // NS
    off = sid * per
    done = 0
    while done < per:
        step = min(CHUNK, per - done)
        pltpu.sync_copy(zslab.at[pl.ds(0, step)],
                        buf.at[pl.ds(off + done, step)])
        done += step


def _pipeline(s1d, d1d, table, buf, slots, w, n_chunks):
    base = w * n_chunks
    n_pairs = n_chunks // 2
    slot0, slot1 = slots

    def its_start(t, row):
        ts, _, _, st, _, _, _ = t
        off = pl.multiple_of((base + row) * CHUNK, 8)
        pltpu.async_copy(s1d.at[pl.ds(off, CHUNK)], ts, st)

    def its_wait(t):
        ts, _, _, st, _, _, _ = t
        pltpu.make_async_copy(s1d.at[pl.ds(0, CHUNK)], ts, st).wait()

    def itd_start(t, row):
        _, td, _, _, sd, _, _ = t
        off = pl.multiple_of((base + row) * CHUNK, 8)
        pltpu.async_copy(d1d.at[pl.ds(off, CHUNK)], td, sd)

    def itd_wait(t):
        _, td, _, _, sd, _, _ = t
        pltpu.make_async_copy(d1d.at[pl.ds(0, CHUNK)], td, sd).wait()

    def g_start(t):
        ts, _, rows, _, _, sg, _ = t
        pltpu.async_copy(table.at[ts], rows, sg)

    def g_wait(t):
        ts, _, rows, _, _, sg, _ = t
        pltpu.make_async_copy(table.at[ts], rows, sg).wait()

    def s_start(t):
        _, td, rows, _, _, _, ss = t
        pltpu.async_copy(rows, buf.at[td], ss, add=True)

    def s_wait(t):
        _, td, rows, _, _, _, ss = t
        pltpu.make_async_copy(rows, buf.at[td], ss).wait()

    its_start(slot0, 0)
    itd_start(slot0, 0)
    its_wait(slot0)
    g_start(slot0)

    def body(p, _):
        a = p * 2

        @pl.when(p > 0)
        def _():
            s_wait(slot1)

        its_start(slot1, a + 1)
        itd_start(slot1, a + 1)
        g_wait(slot0)

        @pl.when(p < n_pairs - 1)
        def _():
            its_start(slot0, a + 2)

        its_wait(slot1)
        g_start(slot1)
        itd_wait(slot0)
        s_start(slot0)
        g_wait(slot1)
        s_wait(slot0)

        @pl.when(p < n_pairs - 1)
        def _():
            itd_start(slot0, a + 2)
            its_wait(slot0)
            g_start(slot0)

        itd_wait(slot1)
        s_start(slot1)
        return 0

    lax.fori_loop(0, n_pairs, body, 0)
    s_wait(slot1)


def _flush(buf, out, cid, sid):
    rows_per = N_OUT // NS
    off = sid * rows_per
    pltpu.sync_copy(buf.at[pl.ds(off, rows_per)],
                    out.at[cid, pl.ds(off, rows_per)])


_SC_MESH = plsc.VectorSubcoreMesh(core_axis_name="c", subcore_axis_name="s",
                                  num_cores=NC, num_subcores=NS)

_SC_SCRATCH = [
    pltpu.VMEM_SHARED((N_BUF, H), jnp.float32),
    pltpu.VMEM((CHUNK, H), jnp.float32),
    pltpu.VMEM((CHUNK,), jnp.int32),
    pltpu.VMEM((CHUNK,), jnp.int32),
    pltpu.VMEM((CHUNK,), jnp.int32),
    pltpu.VMEM((CHUNK,), jnp.int32),
    pltpu.VMEM((CHUNK, H), jnp.float32),
    pltpu.VMEM((CHUNK, H), jnp.float32),
    pltpu.SemaphoreType.DMA,
    pltpu.SemaphoreType.DMA,
    pltpu.SemaphoreType.DMA,
    pltpu.SemaphoreType.DMA,
    pltpu.SemaphoreType.DMA,
    pltpu.SemaphoreType.DMA,
    pltpu.SemaphoreType.DMA,
    pltpu.SemaphoreType.DMA,
]


def _make_slots(refs):
    (zslab, ts0, td0, ts1, td1, rows0, rows1,
     st0, sd0, st1, sd1, sg0, sg1, ss0, ss1) = refs
    return zslab, [(ts0, td0, rows0, st0, sd0, sg0, ss0),
                   (ts1, td1, rows1, st1, sd1, sg1, ss1)]


def _sc_layer0(xb, xj, xf, sb, db, sj, dj, sf, df, sjf, djf,
               nb_chunks, nj_chunks, nf_chunks, njf_chunks):
    agg_ty = jax.ShapeDtypeStruct((NC, N_OUT, H), jnp.float32)

    @functools.partial(
        pl.kernel,
        out_type=(agg_ty, agg_ty, agg_ty, agg_ty),
        mesh=_SC_MESH,
        scratch_types=_SC_SCRATCH,
    )
    def k(xb_h, xj_h, xf_h, sb_h, db_h, sj_h, dj_h, sf_h, df_h, sjf_h, djf_h,
          ob, oj, of_, ojf, buf, *refs):
        cid = lax.axis_index("c")
        sid = lax.axis_index("s")
        w = sid * NC + cid
        zslab, slots = _make_slots(refs)
        _zero_slab(zslab)

        def run(s_h, d_h, table, out, n_chunks):
            _zero_buf(buf, sid, zslab)
            plsc.subcore_barrier()
            _pipeline(s_h, d_h, table, buf, slots, w, n_chunks)
            plsc.subcore_barrier()
            _flush(buf, out, cid, sid)
            plsc.subcore_barrier()

        run(sb_h, db_h, xb_h, ob, nb_chunks)
        run(sf_h, df_h, xf_h, of_, nf_chunks)
        run(sj_h, dj_h, xj_h, oj, nj_chunks)
        run(sjf_h, djf_h, xj_h, ojf, njf_chunks)

    return k(xb, xj, xf, sb, db, sj, dj, sf, df, sjf, djf)


def _sc_layer1(xj1, sjf, djf, njf_chunks):
    agg_ty = jax.ShapeDtypeStruct((NC, N_OUT, H), jnp.float32)

    @functools.partial(
        pl.kernel,
        out_type=agg_ty,
        mesh=_SC_MESH,
        scratch_types=_SC_SCRATCH,
    )
    def k(xj_h, s_h, d_h, out, buf, *refs):
        cid = lax.axis_index("c")
        sid = lax.axis_index("s")
        w = sid * NC + cid
        zslab, slots = _make_slots(refs)
        _zero_slab(zslab)
        _zero_buf(buf, sid, zslab)
        plsc.subcore_barrier()
        _pipeline(s_h, d_h, xj_h, buf, slots, w, njf_chunks)
        plsc.subcore_barrier()
        _flush(buf, out, cid, sid)

    return k(xj1, sjf, djf)


def _pad_edges(ei, n_chunks, dst_clamp=False):
    e_pad = NW * n_chunks * CHUNK
    pad = e_pad - ei.shape[1]
    d = jnp.where(ei[1] < 5000, ei[1], GARBAGE) if dst_clamp else ei[1]
    s = jnp.concatenate([ei[0], jnp.zeros((pad,), jnp.int32)])
    d = jnp.concatenate([d, jnp.full((pad,), GARBAGE, jnp.int32)])
    return s, d


def _n_chunks(e):
    per = pl.cdiv(e, NW * CHUNK)
    return ((per + 1) // 2) * 2


def kernel(x_base, x_joint, x_foot, ei_b2j, ei_j2b, ei_j2j, ei_j2f, ei_f2j,
           W_enc, b_enc, W_rel, b_rel, W_root, W_dec, b_dec):
    del ei_j2b

    nb = _n_chunks(ei_b2j.shape[1])
    nj = _n_chunks(ei_j2j.shape[1])
    nf = _n_chunks(ei_f2j.shape[1])
    njf = _n_chunks(ei_j2f.shape[1])
    sb, db = _pad_edges(ei_b2j, nb)
    sj, dj = _pad_edges(ei_j2j, nj, dst_clamp=True)
    sf, df = _pad_edges(ei_f2j, nf)
    sjf, djf = _pad_edges(ei_j2f, njf)

    xb0 = _encode(x_base, W_enc[0], b_enc[0])
    xj0 = _encode(x_joint, W_enc[1], b_enc[1])
    xf0 = _encode(x_foot, W_enc[2], b_enc[2])

    a_b2j, a_j2j, a_f2j, a_j2f = _sc_layer0(
        xb0, xj0, xf0, sb, db, sj, dj, sf, df, sjf, djf, nb, nj, nf, njf)

    ws_j = jnp.stack([W_rel[0, 0], W_rel[0, 2], W_rel[0, 4]])
    wrs_j = jnp.stack([W_root[0, 0], W_root[0, 2], W_root[0, 4]])
    bs_j = jnp.stack([b_rel[0, 0], b_rel[0, 2], b_rel[0, 4]])
    xj1 = _combine_joint(a_b2j, a_j2j, a_f2j, xj0, ws_j, wrs_j, bs_j, 5000)
    xf1 = _combine_foot(a_j2f, xf0, W_rel[0, 3], W_root[0, 3], b_rel[0, 3],
                        5000)

    a2 = _sc_layer1(xj1, sjf, djf, njf)

    wd_pad = jnp.zeros((H, H), jnp.float32).at[:, 0].set(W_dec[:, 0])
    bd_pad = jnp.zeros((H,), jnp.float32).at[0].set(b_dec[0])
    out = _combine_foot_dec(a2, xf1, W_rel[1, 3], W_root[1, 3], b_rel[1, 3],
                            wd_pad, bd_pad, 5000)
    return out[:, 0:1]


# speedup vs baseline: 1.4888x; 1.1014x over previous
import functools

import jax
import jax.numpy as jnp
from jax import lax
from jax.experimental import pallas as pl
from jax.experimental.pallas import tpu as pltpu
from jax.experimental.pallas import tpu_sc as plsc

H = 128
NC, NS = 2, 16
NW = NC * NS
CHUNK = 128
N_OUT = 5120
N_BUF = 5248
GARBAGE = 5184
BLK = 512


def _mm(a, b):
    return jnp.dot(a, b, preferred_element_type=jnp.float32)


def _enc_body(x_ref, w_ref, b_ref, o_ref):
    o_ref[...] = jnp.maximum(_mm(x_ref[...], w_ref[...]) + b_ref[...], 0.0)


def _encode(x, w, b):
    n = x.shape[0]
    return pl.pallas_call(
        _enc_body,
        grid=(pl.cdiv(n, BLK),),
        in_specs=[
            pl.BlockSpec((BLK, H), lambda i: (i, 0)),
            pl.BlockSpec((H, H), lambda i: (0, 0)),
            pl.BlockSpec((1, H), lambda i: (0, 0)),
        ],
        out_specs=pl.BlockSpec((BLK, H), lambda i: (i, 0)),
        out_shape=jax.ShapeDtypeStruct((n, H), jnp.float32),
    )(x, w, b.reshape(1, H))


def _joint_body(ab_ref, aj_ref, af_ref, x_ref, w_ref, wr_ref, b_ref, o_ref):
    acc = _mm(ab_ref[0] + ab_ref[1], w_ref[0])
    acc += _mm(aj_ref[0] + aj_ref[1], w_ref[1])
    acc += _mm(af_ref[0] + af_ref[1], w_ref[2])
    wr = wr_ref[0] + wr_ref[1] + wr_ref[2]
    acc += _mm(x_ref[...], wr)
    acc += b_ref[0:1] + b_ref[1:2] + b_ref[2:3]
    o_ref[...] = jnp.maximum(acc, 0.0)


def _combine_joint(ab, aj, af, x, ws, wrs, bs, n):
    return pl.pallas_call(
        _joint_body,
        grid=(pl.cdiv(n, BLK),),
        in_specs=[
            pl.BlockSpec((2, BLK, H), lambda i: (0, i, 0)),
            pl.BlockSpec((2, BLK, H), lambda i: (0, i, 0)),
            pl.BlockSpec((2, BLK, H), lambda i: (0, i, 0)),
            pl.BlockSpec((BLK, H), lambda i: (i, 0)),
            pl.BlockSpec((3, H, H), lambda i: (0, 0, 0)),
            pl.BlockSpec((3, H, H), lambda i: (0, 0, 0)),
            pl.BlockSpec((3, H), lambda i: (0, 0)),
        ],
        out_specs=pl.BlockSpec((BLK, H), lambda i: (i, 0)),
        out_shape=jax.ShapeDtypeStruct((n, H), jnp.float32),
    )(ab, aj, af, x, ws, wrs, bs)


def _foot_body(a_ref, x_ref, w_ref, wr_ref, b_ref, o_ref):
    acc = _mm(a_ref[0] + a_ref[1], w_ref[...])
    acc += _mm(x_ref[...], wr_ref[...])
    acc += b_ref[...]
    o_ref[...] = jnp.maximum(acc, 0.0)


def _combine_foot(a, x, w, wr, b, n):
    return pl.pallas_call(
        _foot_body,
        grid=(pl.cdiv(n, BLK),),
        in_specs=[
            pl.BlockSpec((2, BLK, H), lambda i: (0, i, 0)),
            pl.BlockSpec((BLK, H), lambda i: (i, 0)),
            pl.BlockSpec((H, H), lambda i: (0, 0)),
            pl.BlockSpec((H, H), lambda i: (0, 0)),
            pl.BlockSpec((1, H), lambda i: (0, 0)),
        ],
        out_specs=pl.BlockSpec((BLK, H), lambda i: (i, 0)),
        out_shape=jax.ShapeDtypeStruct((n, H), jnp.float32),
    )(a, x, w, wr, b.reshape(1, H))


def _foot_dec_body(a_ref, x_ref, w_ref, wr_ref, b_ref, wd_ref, bd_ref, o_ref):
    acc = _mm(a_ref[0] + a_ref[1], w_ref[...])
    acc += _mm(x_ref[...], wr_ref[...])
    acc += b_ref[...]
    h = jnp.maximum(acc, 0.0)
    o_ref[...] = _mm(h, wd_ref[...]) + bd_ref[...]


def _combine_foot_dec(a, x, w, wr, b, wd, bd, n):
    return pl.pallas_call(
        _foot_dec_body,
        grid=(pl.cdiv(n, BLK),),
        in_specs=[
            pl.BlockSpec((2, BLK, H), lambda i: (0, i, 0)),
            pl.BlockSpec((BLK, H), lambda i: (i, 0)),
            pl.BlockSpec((H, H), lambda i: (0, 0)),
            pl.BlockSpec((H, H), lambda i: (0, 0)),
            pl.BlockSpec((1, H), lambda i: (0, 0)),
            pl.BlockSpec((H, H), lambda i: (0, 0)),
            pl.BlockSpec((1, H), lambda i: (0, 0)),
        ],
        out_specs=pl.BlockSpec((BLK, H), lambda i: (i, 0)),
        out_shape=jax.ShapeDtypeStruct((n, H), jnp.float32),
    )(a, x, w, wr, b.reshape(1, H), wd, bd.reshape(1, H))


def _zero_slab(ref):
    zero = jnp.zeros((16,), jnp.float32)

    def row(i, _):
        for j in range(H // 16):
            ref[i, pl.ds(j * 16, 16)] = zero
        return 0

    lax.fori_loop(0, CHUNK, row, 0)


def _zero_buf(buf, sid, zslab):
    per = N_BUF // NS
    off = sid * per
    done = 0
    while done < per:
        step = min(CHUNK, per - done)
        pltpu.sync_copy(zslab.at[pl.ds(0, step)],
                        buf.at[pl.ds(off + done, step)])
        done += step


def _process(s_ref, d_ref, table, buf, idx_s, idx_d, rows, sem, w, n_chunks):
    base = w * n_chunks * CHUNK

    def body(i, _):
        off = pl.multiple_of(base + i * CHUNK, 8)
        pltpu.sync_copy(s_ref.at[pl.ds(off, CHUNK)], idx_s)
        pltpu.sync_copy(d_ref.at[pl.ds(off, CHUNK)], idx_d)
        pltpu.async_copy(table.at[idx_s], rows, sem).wait()
        pltpu.sync_copy(rows, buf.at[idx_d], add=True)
        return 0

    lax.fori_loop(0, n_chunks, body, 0)


def _flush(buf, out, cid, sid):
    rows_per = N_OUT // NS
    off = sid * rows_per
    pltpu.sync_copy(buf.at[pl.ds(off, rows_per)],
                    out.at[cid, pl.ds(off, rows_per)])


_SC_MESH = plsc.VectorSubcoreMesh(core_axis_name="c", subcore_axis_name="s",
                                  num_cores=NC, num_subcores=NS)

_SC_SCRATCH = [
    pltpu.VMEM_SHARED((N_BUF, H), jnp.float32),
    pltpu.VMEM_SHARED((N_BUF, H), jnp.float32),
    pltpu.VMEM((CHUNK,), jnp.int32),
    pltpu.VMEM((CHUNK,), jnp.int32),
    pltpu.VMEM((CHUNK, H), jnp.float32),
    pltpu.VMEM((CHUNK, H), jnp.float32),
    pltpu.SemaphoreType.DMA,
]


def _sc_layer0(xb, xj, xf, sb, db, sj, dj, sf, df, sjf, djf,
               nb_chunks, nj_chunks, nf_chunks, njf_chunks):
    agg_ty = jax.ShapeDtypeStruct((NC, N_OUT, H), jnp.float32)

    @functools.partial(
        pl.kernel,
        out_type=(agg_ty, agg_ty, agg_ty, agg_ty),
        mesh=_SC_MESH,
        scratch_types=_SC_SCRATCH,
    )
    def k(xb_h, xj_h, xf_h, sb_h, db_h, sj_h, dj_h, sf_h, df_h, sjf_h, djf_h,
          ob, oj, of_, ojf, bufA, bufB, idx_s, idx_d, rows, zslab, sem):
        cid = lax.axis_index("c")
        sid = lax.axis_index("s")
        w = sid * NC + cid
        _zero_slab(zslab)
        _zero_buf(bufA, sid, zslab)
        _zero_buf(bufB, sid, zslab)
        plsc.subcore_barrier()
        _process(sb_h, db_h, xb_h, bufA, idx_s, idx_d, rows, sem, w,
                 nb_chunks)
        _process(sj_h, dj_h, xj_h, bufB, idx_s, idx_d, rows, sem, w,
                 nj_chunks)
        plsc.subcore_barrier()
        _flush(bufA, ob, cid, sid)
        _flush(bufB, oj, cid, sid)
        plsc.subcore_barrier()
        _zero_buf(bufA, sid, zslab)
        _zero_buf(bufB, sid, zslab)
        plsc.subcore_barrier()
        _process(sf_h, df_h, xf_h, bufA, idx_s, idx_d, rows, sem, w,
                 nf_chunks)
        _process(sjf_h, djf_h, xj_h, bufB, idx_s, idx_d, rows, sem, w,
                 njf_chunks)
        plsc.subcore_barrier()
        _flush(bufA, of_, cid, sid)
        _flush(bufB, ojf, cid, sid)

    return k(xb, xj, xf, sb, db, sj, dj, sf, df, sjf, djf)


def _sc_layer1(xj1, sjf, djf, njf_chunks):
    agg_ty = jax.ShapeDtypeStruct((NC, N_OUT, H), jnp.float32)

    @functools.partial(
        pl.kernel,
        out_type=agg_ty,
        mesh=_SC_MESH,
        scratch_types=_SC_SCRATCH,
    )
    def k(xj_h, s_h, d_h, out, bufA, bufB, idx_s, idx_d, rows, zslab, sem):
        cid = lax.axis_index("c")
        sid = lax.axis_index("s")
        w = sid * NC + cid
        _zero_slab(zslab)
        _zero_buf(bufA, sid, zslab)
        plsc.subcore_barrier()
        _process(s_h, d_h, xj_h, bufA, idx_s, idx_d, rows, sem, w,
                 njf_chunks)
        plsc.subcore_barrier()
        _flush(bufA, out, cid, sid)

    return k(xj1, sjf, djf)


def _pad_edges(ei, n_chunks, dst_clamp=False):
    e_pad = NW * n_chunks * CHUNK
    pad = e_pad - ei.shape[1]
    d = jnp.where(ei[1] < 5000, ei[1], GARBAGE) if dst_clamp else ei[1]
    s = jnp.concatenate([ei[0], jnp.zeros((pad,), jnp.int32)])
    d = jnp.concatenate([d, jnp.full((pad,), GARBAGE, jnp.int32)])
    return s, d


def _n_chunks(e):
    return pl.cdiv(e, NW * CHUNK)


def kernel(x_base, x_joint, x_foot, ei_b2j, ei_j2b, ei_j2j, ei_j2f, ei_f2j,
           W_enc, b_enc, W_rel, b_rel, W_root, W_dec, b_dec):
    del ei_j2b

    nb = _n_chunks(ei_b2j.shape[1])
    nj = _n_chunks(ei_j2j.shape[1])
    nf = _n_chunks(ei_f2j.shape[1])
    njf = _n_chunks(ei_j2f.shape[1])
    sb, db = _pad_edges(ei_b2j, nb)
    sj, dj = _pad_edges(ei_j2j, nj, dst_clamp=True)
    sf, df = _pad_edges(ei_f2j, nf)
    sjf, djf = _pad_edges(ei_j2f, njf)

    xb0 = _encode(x_base, W_enc[0], b_enc[0])
    xj0 = _encode(x_joint, W_enc[1], b_enc[1])
    xf0 = _encode(x_foot, W_enc[2], b_enc[2])

    a_b2j, a_j2j, a_f2j, a_j2f = _sc_layer0(
        xb0, xj0, xf0, sb, db, sj, dj, sf, df, sjf, djf, nb, nj, nf, njf)

    ws_j = jnp.stack([W_rel[0, 0], W_rel[0, 2], W_rel[0, 4]])
    wrs_j = jnp.stack([W_root[0, 0], W_root[0, 2], W_root[0, 4]])
    bs_j = jnp.stack([b_rel[0, 0], b_rel[0, 2], b_rel[0, 4]])
    xj1 = _combine_joint(a_b2j, a_j2j, a_f2j, xj0, ws_j, wrs_j, bs_j, 5000)
    xf1 = _combine_foot(a_j2f, xf0, W_rel[0, 3], W_root[0, 3], b_rel[0, 3],
                        5000)

    a2 = _sc_layer1(xj1, sjf, djf, njf)

    wd_pad = jnp.zeros((H, H), jnp.float32).at[:, 0].set(W_dec[:, 0])
    bd_pad = jnp.zeros((H,), jnp.float32).at[0].set(b_dec[0])
    out = _combine_foot_dec(a2, xf1, W_rel[1, 3], W_root[1, 3], b_rel[1, 3],
                            wd_pad, bd_pad, 5000)
    return out[:, 0:1]
